# Initial kernel scaffold; baseline (speedup 1.0000x reference)
import jax
import jax.numpy as jnp
from jax.experimental import pallas as pl


def kernel(semantic_ids_l1, semantic_ids_l2, semantic_ids_l3, E1, E2, E3, W, b):
    raise NotImplementedError("write your pallas kernel here")


# speedup vs baseline: 4.4492x; 4.4492x over previous
import functools

import jax
import jax.numpy as jnp
from jax import lax
from jax.experimental import pallas as pl
from jax.experimental.pallas import tpu as pltpu
from jax.experimental.pallas import tpu_sc as plsc

D_OUT = 256
CHUNK = 128


def _proj_body(x_ref, wt_ref, b_ref, o_ref):
    o_ref[...] = (
        jnp.dot(x_ref[...], wt_ref[...], preferred_element_type=jnp.float32)
        + b_ref[...]
    )


def _project_table(E, Wt, bias_row):
    V, K = E.shape
    BV = min(V, 1024)
    return pl.pallas_call(
        _proj_body,
        grid=(V // BV,),
        in_specs=[
            pl.BlockSpec((BV, K), lambda i: (i, 0)),
            pl.BlockSpec((K, D_OUT), lambda i: (0, 0)),
            pl.BlockSpec((1, D_OUT), lambda i: (0, 0)),
        ],
        out_specs=pl.BlockSpec((BV, D_OUT), lambda i: (i, 0)),
        out_shape=jax.ShapeDtypeStruct((V, D_OUT), jnp.float32),
    )(E, Wt, bias_row)


@functools.lru_cache(maxsize=None)
def _make_sc_gather_sum(n_tokens):
    info = plsc.get_sparse_core_info()
    nc, ns, lanes = info.num_cores, info.num_subcores, info.num_lanes
    nw = nc * ns
    assert n_tokens % (nw * CHUNK) == 0
    per_w = n_tokens // nw
    n_chunks = per_w // CHUNK
    mesh = plsc.VectorSubcoreMesh(core_axis_name="c", subcore_axis_name="s")

    @functools.partial(
        pl.kernel,
        mesh=mesh,
        out_type=jax.ShapeDtypeStruct((n_tokens, D_OUT), jnp.float32),
        scratch_types=[
            pltpu.VMEM((CHUNK,), jnp.int32),
            pltpu.VMEM((CHUNK,), jnp.int32),
            pltpu.VMEM((CHUNK,), jnp.int32),
            pltpu.VMEM((CHUNK, D_OUT), jnp.float32),
            pltpu.VMEM((CHUNK, D_OUT), jnp.float32),
            pltpu.VMEM((CHUNK, D_OUT), jnp.float32),
            pltpu.SemaphoreType.DMA,
        ],
    )
    def sc_gather_sum(i1, i2, i3, p1, p2, p3, out, x1, x2, x3, r1, r2, r3, sem):
        wid = lax.axis_index("s") * nc + lax.axis_index("c")
        base0 = wid * per_w

        def chunk_body(ci, carry):
            base = base0 + ci * CHUNK
            pltpu.sync_copy(i1.at[pl.ds(base, CHUNK)], x1)
            pltpu.sync_copy(i2.at[pl.ds(base, CHUNK)], x2)
            pltpu.sync_copy(i3.at[pl.ds(base, CHUNK)], x3)
            d1 = pltpu.async_copy(p1.at[x1], r1, sem)
            d2 = pltpu.async_copy(p2.at[x2], r2, sem)
            d3 = pltpu.async_copy(p3.at[x3], r3, sem)
            d1.wait()
            d2.wait()
            d3.wait()

            def tok_body(t, c2):
                for j in range(D_OUT // lanes):
                    sl = pl.ds(j * lanes, lanes)
                    r3[t, sl] = r3[t, sl] + r1[t, sl] + r2[t, sl]
                return c2

            lax.fori_loop(0, CHUNK, tok_body, 0)
            pltpu.sync_copy(r3, out.at[pl.ds(base, CHUNK)])
            return carry

        lax.fori_loop(0, n_chunks, chunk_body, 0)

    return sc_gather_sum


def kernel(semantic_ids_l1, semantic_ids_l2, semantic_ids_l3, E1, E2, E3, W, b):
    B, S = semantic_ids_l1.shape
    n = B * S
    i1 = semantic_ids_l1.reshape(n).astype(jnp.int32)
    i2 = semantic_ids_l2.reshape(n).astype(jnp.int32)
    i3 = semantic_ids_l3.reshape(n).astype(jnp.int32)

    k1 = E1.shape[1]
    k2 = E2.shape[1]
    Wt = W.T
    zero_row = jnp.zeros((1, D_OUT), dtype=jnp.float32)
    P1 = _project_table(E1, Wt[:k1], b.reshape(1, D_OUT))
    P2 = _project_table(E2, Wt[k1 : k1 + k2], zero_row)
    P3 = _project_table(E3, Wt[k1 + k2 :], zero_row)

    out = _make_sc_gather_sum(n)(i1, i2, i3, P1, P2, P3)
    return out.reshape(B, S, D_OUT)


# speedup vs baseline: 6.8912x; 1.5489x over previous
import functools

import jax
import jax.numpy as jnp
from jax import lax
from jax.experimental import pallas as pl
from jax.experimental.pallas import tpu as pltpu
from jax.experimental.pallas import tpu_sc as plsc

D_OUT = 256
CHUNK = 40


def _proj_body(x_ref, wt_ref, b_ref, o_ref):
    o_ref[...] = (
        jnp.dot(x_ref[...], wt_ref[...], preferred_element_type=jnp.float32)
        + b_ref[...]
    )


def _project_table(E, Wt, bias_row):
    V, K = E.shape
    BV = min(V, 1024)
    return pl.pallas_call(
        _proj_body,
        grid=(V // BV,),
        in_specs=[
            pl.BlockSpec((BV, K), lambda i: (i, 0)),
            pl.BlockSpec((K, D_OUT), lambda i: (0, 0)),
            pl.BlockSpec((1, D_OUT), lambda i: (0, 0)),
        ],
        out_specs=pl.BlockSpec((BV, D_OUT), lambda i: (i, 0)),
        out_shape=jax.ShapeDtypeStruct((V, D_OUT), jnp.float32),
    )(E, Wt, bias_row)


@functools.lru_cache(maxsize=None)
def _make_sc_gather_sum(n_tokens):
    info = plsc.get_sparse_core_info()
    nc, ns, lanes = info.num_cores, info.num_subcores, info.num_lanes
    nw = nc * ns
    assert n_tokens % (nw * CHUNK) == 0
    per_w = n_tokens // nw
    n_chunks = per_w // CHUNK
    assert n_chunks % 2 == 0 and n_chunks >= 6
    n_outer = n_chunks // 2
    mesh = plsc.VectorSubcoreMesh(core_axis_name="c", subcore_axis_name="s")
    row_f32 = pltpu.VMEM((CHUNK, D_OUT), jnp.float32)

    @functools.partial(
        pl.kernel,
        mesh=mesh,
        out_type=jax.ShapeDtypeStruct((n_tokens, D_OUT), jnp.float32),
        scratch_types=[
            pltpu.VMEM((3, CHUNK), jnp.int32),
            pltpu.VMEM((3, CHUNK), jnp.int32),
            row_f32, row_f32, row_f32, row_f32,
            row_f32, row_f32, row_f32, row_f32,
            pltpu.SemaphoreType.DMA, pltpu.SemaphoreType.DMA,
            pltpu.SemaphoreType.DMA, pltpu.SemaphoreType.DMA,
        ],
    )
    def sc_gather_sum(idsp, p1, p2, p3, out,
                      x0, x1, r1a, r1b, r2a, r2b, r3a, r3b, oa, ob,
                      gsem0, gsem1, osem0, osem1):
        x = (x0, x1)
        r1 = (r1a, r1b)
        r2 = (r2a, r2b)
        r3 = (r3a, r3b)
        o = (oa, ob)
        gsem = (gsem0, gsem1)
        osem = (osem0, osem1)
        wid = lax.axis_index("s") * nc + lax.axis_index("c")
        base0 = wid * per_w
        chunk0 = wid * n_chunks

        def load_idx_and_fire(b, ci):
            pltpu.sync_copy(idsp.at[chunk0 + ci], x[b])
            pltpu.async_copy(p1.at[x[b].at[0]], r1[b], gsem[b])
            pltpu.async_copy(p2.at[x[b].at[1]], r2[b], gsem[b])
            pltpu.async_copy(p3.at[x[b].at[2]], r3[b], gsem[b])

        def wait_gathers(b):
            pltpu.make_async_copy(p1.at[x[b].at[0]], r1[b], gsem[b]).wait()
            pltpu.make_async_copy(p2.at[x[b].at[1]], r2[b], gsem[b]).wait()
            pltpu.make_async_copy(p3.at[x[b].at[2]], r3[b], gsem[b]).wait()

        def compute(b):
            def tok(t, c):
                for j in range(D_OUT // lanes):
                    sl = pl.ds(j * lanes, lanes)
                    o[b][t, sl] = r1[b][t, sl] + r2[b][t, sl] + r3[b][t, sl]
                return c

            lax.fori_loop(0, CHUNK, tok, 0)

        def fire_store(b, ci):
            dst = out.at[pl.ds(base0 + ci * CHUNK, CHUNK)]
            pltpu.async_copy(o[b], dst, osem[b])

        def wait_store(b):
            dst = out.at[pl.ds(base0, CHUNK)]
            pltpu.make_async_copy(o[b], dst, osem[b]).wait()

        for b in range(2):
            load_idx_and_fire(b, b)
        for b in range(2):
            wait_gathers(b)
            compute(b)
            fire_store(b, b)
            load_idx_and_fire(b, b + 2)

        def outer(i, c):
            for b in range(2):
                ci = 2 * i + b
                wait_gathers(b)
                wait_store(b)
                compute(b)
                fire_store(b, ci)
                load_idx_and_fire(b, ci + 2)
            return c

        lax.fori_loop(1, n_outer - 1, outer, 0)

        for b in range(2):
            ci = n_chunks - 2 + b
            wait_gathers(b)
            wait_store(b)
            compute(b)
            fire_store(b, ci)
        for b in range(2):
            wait_store(b)

    return sc_gather_sum


def kernel(semantic_ids_l1, semantic_ids_l2, semantic_ids_l3, E1, E2, E3, W, b):
    B, S = semantic_ids_l1.shape
    n = B * S
    i1 = semantic_ids_l1.reshape(n).astype(jnp.int32)
    i2 = semantic_ids_l2.reshape(n).astype(jnp.int32)
    i3 = semantic_ids_l3.reshape(n).astype(jnp.int32)
    idsp = (
        jnp.stack([i1, i2, i3])
        .reshape(3, n // CHUNK, CHUNK)
        .transpose(1, 0, 2)
    )

    k1 = E1.shape[1]
    k2 = E2.shape[1]
    Wt = W.T
    zero_row = jnp.zeros((1, D_OUT), dtype=jnp.float32)
    P1 = _project_table(E1, Wt[:k1], b.reshape(1, D_OUT))
    P2 = _project_table(E2, Wt[k1 : k1 + k2], zero_row)
    P3 = _project_table(E3, Wt[k1 + k2 :], zero_row)

    out = _make_sc_gather_sum(n)(idsp, P1, P2, P3)
    return out.reshape(B, S, D_OUT)


# speedup vs baseline: 7.8434x; 1.1382x over previous
import functools

import jax
import jax.numpy as jnp
import numpy as np
from jax import lax
from jax.experimental import pallas as pl
from jax.experimental.pallas import tpu as pltpu
from jax.experimental.pallas import tpu_sc as plsc

D_OUT = 256
CHUNK = 40
NBUF = 4


def _interleave_perm():
    perm = np.empty((D_OUT,), dtype=np.int64)
    for g in range(D_OUT // 32):
        for i in range(16):
            perm[32 * g + 2 * i] = 32 * g + i
            perm[32 * g + 2 * i + 1] = 32 * g + 16 + i
    return perm


def _proj_body(x_ref, wt_ref, b_ref, o_ref):
    o_ref[...] = (
        jnp.dot(x_ref[...], wt_ref[...], preferred_element_type=jnp.float32)
        + b_ref[...]
    ).astype(jnp.bfloat16)


def _project_table(E, Wt, bias_row):
    V, K = E.shape
    BV = min(V, 1024)
    return pl.pallas_call(
        _proj_body,
        grid=(V // BV,),
        in_specs=[
            pl.BlockSpec((BV, K), lambda i: (i, 0)),
            pl.BlockSpec((K, D_OUT), lambda i: (0, 0)),
            pl.BlockSpec((1, D_OUT), lambda i: (0, 0)),
        ],
        out_specs=pl.BlockSpec((BV, D_OUT), lambda i: (i, 0)),
        out_shape=jax.ShapeDtypeStruct((V, D_OUT), jnp.bfloat16),
    )(E, Wt, bias_row)


@functools.lru_cache(maxsize=None)
def _make_sc_gather_sum(n_tokens):
    info = plsc.get_sparse_core_info()
    nc, ns, lanes = info.num_cores, info.num_subcores, info.num_lanes
    nw = nc * ns
    assert n_tokens % (nw * CHUNK) == 0
    per_w = n_tokens // nw
    n_chunks = per_w // CHUNK
    assert n_chunks % NBUF == 0 and n_chunks >= 3 * NBUF
    n_outer = n_chunks // NBUF
    mesh = plsc.VectorSubcoreMesh(core_axis_name="c", subcore_axis_name="s")
    row_f32 = pltpu.VMEM((CHUNK, D_OUT), jnp.float32)
    row_i32 = pltpu.VMEM((CHUNK, D_OUT // 2), jnp.int32)

    @functools.partial(
        pl.kernel,
        mesh=mesh,
        out_type=jax.ShapeDtypeStruct((n_tokens, D_OUT), jnp.float32),
        scratch_types=(
            [pltpu.VMEM((3, CHUNK), jnp.int32)] * NBUF
            + [row_i32] * (3 * NBUF)
            + [row_f32] * NBUF
            + [pltpu.SemaphoreType.DMA] * (3 * NBUF)
        ),
    )
    def sc_gather_sum(idsp, p1, p2, p3, out, *bufs):
        x = bufs[0:NBUF]
        r1 = bufs[NBUF : 2 * NBUF]
        r2 = bufs[2 * NBUF : 3 * NBUF]
        r3 = bufs[3 * NBUF : 4 * NBUF]
        o = bufs[4 * NBUF : 5 * NBUF]
        gsem = bufs[5 * NBUF : 6 * NBUF]
        osem = bufs[6 * NBUF : 7 * NBUF]
        isem = bufs[7 * NBUF : 8 * NBUF]
        wid = lax.axis_index("s") * nc + lax.axis_index("c")
        base0 = wid * per_w
        chunk0 = wid * n_chunks

        def fire_idx(b, ci):
            pltpu.async_copy(idsp.at[chunk0 + ci], x[b], isem[b])

        def wait_idx(b, ci):
            pltpu.make_async_copy(idsp.at[chunk0 + ci], x[b], isem[b]).wait()

        def fire_gathers(b):
            pltpu.async_copy(p1.at[x[b].at[0]], r1[b], gsem[b])
            pltpu.async_copy(p2.at[x[b].at[1]], r2[b], gsem[b])
            pltpu.async_copy(p3.at[x[b].at[2]], r3[b], gsem[b])

        def wait_gathers(b):
            pltpu.make_async_copy(p1.at[x[b].at[0]], r1[b], gsem[b]).wait()
            pltpu.make_async_copy(p2.at[x[b].at[1]], r2[b], gsem[b]).wait()
            pltpu.make_async_copy(p3.at[x[b].at[2]], r3[b], gsem[b]).wait()

        def compute(b):
            himask = jnp.int32(-65536)

            def as_f32_pair(v):
                return (
                    lax.bitcast_convert_type(v << 16, jnp.float32),
                    lax.bitcast_convert_type(v & himask, jnp.float32),
                )

            @plsc.parallel_loop(0, CHUNK, step=1, unroll=4)
            def tok(t):
                for g in range(D_OUT // 32):
                    sl = pl.ds(g * lanes, lanes)
                    lo1, hi1 = as_f32_pair(r1[b][t, sl])
                    lo2, hi2 = as_f32_pair(r2[b][t, sl])
                    lo3, hi3 = as_f32_pair(r3[b][t, sl])
                    o[b][t, pl.ds(g * 32, 16)] = lo1 + lo2 + lo3
                    o[b][t, pl.ds(g * 32 + 16, 16)] = hi1 + hi2 + hi3

        def fire_store(b, ci):
            dst = out.at[pl.ds(base0 + ci * CHUNK, CHUNK)]
            pltpu.async_copy(o[b], dst, osem[b])

        def wait_store(b):
            dst = out.at[pl.ds(base0, CHUNK)]
            pltpu.make_async_copy(o[b], dst, osem[b]).wait()

        for b in range(NBUF):
            fire_idx(b, b)
            wait_idx(b, b)
            fire_gathers(b)
        for b in range(NBUF):
            wait_gathers(b)
            fire_idx(b, b + NBUF)
            compute(b)
            fire_store(b, b)
            wait_idx(b, b + NBUF)
            fire_gathers(b)

        def outer(i, c):
            for b in range(NBUF):
                ci = NBUF * i + b
                wait_gathers(b)
                fire_idx(b, ci + NBUF)
                wait_store(b)
                compute(b)
                fire_store(b, ci)
                wait_idx(b, ci + NBUF)
                fire_gathers(b)
            return c

        lax.fori_loop(1, n_outer - 1, outer, 0)

        for b in range(NBUF):
            ci = n_chunks - NBUF + b
            wait_gathers(b)
            wait_store(b)
            compute(b)
            fire_store(b, ci)
        for b in range(NBUF):
            wait_store(b)

    return sc_gather_sum


def kernel(semantic_ids_l1, semantic_ids_l2, semantic_ids_l3, E1, E2, E3, W, b):
    B, S = semantic_ids_l1.shape
    n = B * S
    i1 = semantic_ids_l1.reshape(n).astype(jnp.int32)
    i2 = semantic_ids_l2.reshape(n).astype(jnp.int32)
    i3 = semantic_ids_l3.reshape(n).astype(jnp.int32)
    idsp = (
        jnp.stack([i1, i2, i3])
        .reshape(3, n // CHUNK, CHUNK)
        .transpose(1, 0, 2)
    )

    k1 = E1.shape[1]
    k2 = E2.shape[1]
    perm = _interleave_perm()
    Wt = W.T[:, perm]
    zero_row = jnp.zeros((1, D_OUT), dtype=jnp.float32)
    def _pack_i32(P):
        V = P.shape[0]
        return lax.bitcast_convert_type(
            P.reshape(V, D_OUT // 2, 2), jnp.int32
        )

    P1 = _pack_i32(_project_table(E1, Wt[:k1], b[perm].reshape(1, D_OUT)))
    P2 = _pack_i32(_project_table(E2, Wt[k1 : k1 + k2], zero_row))
    P3 = _pack_i32(_project_table(E3, Wt[k1 + k2 :], zero_row))

    out = _make_sc_gather_sum(n)(idsp, P1, P2, P3)
    return out.reshape(B, S, D_OUT)


# speedup vs baseline: 8.1591x; 1.0403x over previous
import functools

import jax
import jax.numpy as jnp
import numpy as np
from jax import lax
from jax.experimental import pallas as pl
from jax.experimental.pallas import tpu as pltpu
from jax.experimental.pallas import tpu_sc as plsc

D_OUT = 256
CHUNK = 80


def _interleave_perm():
    perm = np.empty((D_OUT,), dtype=np.int64)
    for g in range(D_OUT // 32):
        for i in range(16):
            perm[32 * g + 2 * i] = 32 * g + i
            perm[32 * g + 2 * i + 1] = 32 * g + 16 + i
    return perm


def _proj_body(x_ref, wt_ref, b_ref, o_ref):
    o_ref[...] = (
        jnp.dot(x_ref[...], wt_ref[...], preferred_element_type=jnp.float32)
        + b_ref[...]
    ).astype(jnp.bfloat16)


def _project_table(E, Wt, bias_row):
    V, K = E.shape
    BV = min(V, 1024)
    return pl.pallas_call(
        _proj_body,
        grid=(V // BV,),
        in_specs=[
            pl.BlockSpec((BV, K), lambda i: (i, 0)),
            pl.BlockSpec((K, D_OUT), lambda i: (0, 0)),
            pl.BlockSpec((1, D_OUT), lambda i: (0, 0)),
        ],
        out_specs=pl.BlockSpec((BV, D_OUT), lambda i: (i, 0)),
        out_shape=jax.ShapeDtypeStruct((V, D_OUT), jnp.bfloat16),
    )(E, Wt, bias_row)


@functools.lru_cache(maxsize=None)
def _make_sc_gather_sum(n_tokens):
    info = plsc.get_sparse_core_info()
    nc, ns, lanes = info.num_cores, info.num_subcores, info.num_lanes
    nw = nc * ns
    assert n_tokens % (nw * CHUNK) == 0
    per_w = n_tokens // nw
    n_chunks = per_w // CHUNK
    assert n_chunks % 2 == 0 and n_chunks >= 6
    n_outer = n_chunks // 2
    mesh = plsc.VectorSubcoreMesh(core_axis_name="c", subcore_axis_name="s")
    row_f32 = pltpu.VMEM((CHUNK, D_OUT), jnp.float32)
    row_i32 = pltpu.VMEM((CHUNK, D_OUT // 2), jnp.int32)

    @functools.partial(
        pl.kernel,
        mesh=mesh,
        out_type=jax.ShapeDtypeStruct((n_tokens, D_OUT), jnp.float32),
        scratch_types=[
            pltpu.VMEM((3, CHUNK), jnp.int32),
            pltpu.VMEM((3, CHUNK), jnp.int32),
            row_i32, row_i32, row_i32, row_i32,
            row_i32, row_i32, row_f32, row_f32,
            pltpu.SemaphoreType.DMA, pltpu.SemaphoreType.DMA,
            pltpu.SemaphoreType.DMA, pltpu.SemaphoreType.DMA,
            pltpu.SemaphoreType.DMA, pltpu.SemaphoreType.DMA,
        ],
    )
    def sc_gather_sum(idsp, p1, p2, p3, out,
                      x0, x1, r1a, r1b, r2a, r2b, r3a, r3b, oa, ob,
                      gsem0, gsem1, osem0, osem1, isem0, isem1):
        x = (x0, x1)
        r1 = (r1a, r1b)
        r2 = (r2a, r2b)
        r3 = (r3a, r3b)
        o = (oa, ob)
        gsem = (gsem0, gsem1)
        osem = (osem0, osem1)
        isem = (isem0, isem1)
        wid = lax.axis_index("s") * nc + lax.axis_index("c")
        base0 = wid * per_w
        chunk0 = wid * n_chunks

        def fire_idx(b, ci):
            pltpu.async_copy(idsp.at[chunk0 + ci], x[b], isem[b])

        def wait_idx(b, ci):
            pltpu.make_async_copy(idsp.at[chunk0 + ci], x[b], isem[b]).wait()

        def fire_gathers(b):
            pltpu.async_copy(p1.at[x[b].at[0]], r1[b], gsem[b])
            pltpu.async_copy(p2.at[x[b].at[1]], r2[b], gsem[b])
            pltpu.async_copy(p3.at[x[b].at[2]], r3[b], gsem[b])

        def wait_gathers(b):
            pltpu.make_async_copy(p1.at[x[b].at[0]], r1[b], gsem[b]).wait()
            pltpu.make_async_copy(p2.at[x[b].at[1]], r2[b], gsem[b]).wait()
            pltpu.make_async_copy(p3.at[x[b].at[2]], r3[b], gsem[b]).wait()

        def compute(b):
            himask = jnp.int32(-65536)

            def as_f32_pair(v):
                return (
                    lax.bitcast_convert_type(v << 16, jnp.float32),
                    lax.bitcast_convert_type(v & himask, jnp.float32),
                )

            @plsc.parallel_loop(0, CHUNK, step=1, unroll=4)
            def tok(t):
                for g in range(D_OUT // 32):
                    sl = pl.ds(g * lanes, lanes)
                    lo1, hi1 = as_f32_pair(r1[b][t, sl])
                    lo2, hi2 = as_f32_pair(r2[b][t, sl])
                    lo3, hi3 = as_f32_pair(r3[b][t, sl])
                    o[b][t, pl.ds(g * 32, 16)] = lo1 + lo2 + lo3
                    o[b][t, pl.ds(g * 32 + 16, 16)] = hi1 + hi2 + hi3

        def fire_store(b, ci):
            dst = out.at[pl.ds(base0 + ci * CHUNK, CHUNK)]
            pltpu.async_copy(o[b], dst, osem[b])

        def wait_store(b):
            dst = out.at[pl.ds(base0, CHUNK)]
            pltpu.make_async_copy(o[b], dst, osem[b]).wait()

        for b in range(2):
            fire_idx(b, b)
            wait_idx(b, b)
            fire_gathers(b)
        for b in range(2):
            wait_gathers(b)
            fire_idx(b, b + 2)
            compute(b)
            fire_store(b, b)
            wait_idx(b, b + 2)
            fire_gathers(b)

        def outer(i, c):
            for b in range(2):
                ci = 2 * i + b
                wait_gathers(b)
                fire_idx(b, ci + 2)
                wait_store(b)
                compute(b)
                fire_store(b, ci)
                wait_idx(b, ci + 2)
                fire_gathers(b)
            return c

        lax.fori_loop(1, n_outer - 1, outer, 0)

        for b in range(2):
            ci = n_chunks - 2 + b
            wait_gathers(b)
            wait_store(b)
            compute(b)
            fire_store(b, ci)
        for b in range(2):
            wait_store(b)

    return sc_gather_sum


def kernel(semantic_ids_l1, semantic_ids_l2, semantic_ids_l3, E1, E2, E3, W, b):
    B, S = semantic_ids_l1.shape
    n = B * S
    i1 = semantic_ids_l1.reshape(n).astype(jnp.int32)
    i2 = semantic_ids_l2.reshape(n).astype(jnp.int32)
    i3 = semantic_ids_l3.reshape(n).astype(jnp.int32)
    idsp = (
        jnp.stack([i1, i2, i3])
        .reshape(3, n // CHUNK, CHUNK)
        .transpose(1, 0, 2)
    )

    k1 = E1.shape[1]
    k2 = E2.shape[1]
    perm = _interleave_perm()
    Wt = W.T[:, perm]
    zero_row = jnp.zeros((1, D_OUT), dtype=jnp.float32)
    def _pack_i32(P):
        V = P.shape[0]
        return lax.bitcast_convert_type(
            P.reshape(V, D_OUT // 2, 2), jnp.int32
        )

    P1 = _pack_i32(_project_table(E1, Wt[:k1], b[perm].reshape(1, D_OUT)))
    P2 = _pack_i32(_project_table(E2, Wt[k1 : k1 + k2], zero_row))
    P3 = _pack_i32(_project_table(E3, Wt[k1 + k2 :], zero_row))

    out = _make_sc_gather_sum(n)(idsp, P1, P2, P3)
    return out.reshape(B, S, D_OUT)
